# baseline (speedup 1.0000x reference)
import functools

import jax
import jax.numpy as jnp
from jax import lax
from jax.experimental import pallas as pl
from jax.experimental.pallas import tpu as pltpu
from jax.experimental.pallas import tpu_sc as plsc

NN = 10000
NE = 320000
D = 128
DB = 16
NC = 2
NS = 16
NW = NC * NS
CH = 128
NCH = 80
EPW = CH * NCH
NE_P = NW * EPW
NN_P = 10240
RPW = NN_P // NS


def _mesh():
    return plsc.VectorSubcoreMesh(core_axis_name="c", subcore_axis_name="s")


@functools.partial(jax.jit, static_argnames=())
def _sc_gather(table, idx3):
    @functools.partial(
        pl.kernel,
        out_type=jax.ShapeDtypeStruct((NE_P, D), jnp.float32),
        mesh=_mesh(),
        scratch_types=[
            pltpu.VMEM((NCH, CH), jnp.int32),
            pltpu.VMEM((CH, D), jnp.float32),
            pltpu.SemaphoreType.DMA,
        ],
    )
    def k(table_hbm, idx_hbm, out_hbm, idx_v, rows_v, sem):
        w = lax.axis_index("s") * NC + lax.axis_index("c")
        pltpu.sync_copy(idx_hbm.at[w], idx_v)

        def body(j, carry):
            pltpu.async_copy(table_hbm.at[idx_v.at[j]], rows_v, sem).wait()
            pltpu.sync_copy(rows_v, out_hbm.at[pl.ds(w * EPW + j * CH, CH)])
            return carry

        lax.fori_loop(0, NCH, body, 0)

    return k(table, idx3)


def _sc_scatter(vals, dst3, zrows):
    @functools.partial(
        pl.kernel,
        out_type=jax.ShapeDtypeStruct((NC, NN_P, D), jnp.float32),
        mesh=_mesh(),
        scratch_types=[
            pltpu.VMEM((NCH, CH), jnp.int32),
            pltpu.VMEM((CH, D), jnp.float32),
            pltpu.VMEM_SHARED((NN_P, D), jnp.float32),
        ],
    )
    def k(vals_hbm, idx_hbm, z_hbm, out_hbm, idx_v, rows_v, agg_sh):
        c = lax.axis_index("c")
        s = lax.axis_index("s")
        w = s * NC + c
        pltpu.sync_copy(z_hbm.at[pl.ds(s * RPW, RPW)],
                        agg_sh.at[pl.ds(s * RPW, RPW)])
        pltpu.sync_copy(idx_hbm.at[w], idx_v)
        plsc.subcore_barrier()

        def body(j, carry):
            pltpu.sync_copy(vals_hbm.at[pl.ds(w * EPW + j * CH, CH)], rows_v)
            pltpu.sync_copy(rows_v, agg_sh.at[idx_v.at[j]], add=True)
            return carry

        lax.fori_loop(0, NCH, body, 0)
        plsc.subcore_barrier()
        pltpu.sync_copy(agg_sh.at[pl.ds(s * RPW, RPW)],
                        out_hbm.at[c, pl.ds(s * RPW, RPW)])

    return k(vals, dst3, zrows)


def _tc_g(x, wix, bi2):
    BR = 400

    def body(x_ref, w_ref, b_ref, o_ref):
        o_ref[...] = jnp.dot(x_ref[...], w_ref[...],
                             preferred_element_type=jnp.float32) + b_ref[...]

    return pl.pallas_call(
        body,
        grid=(NN // BR,),
        in_specs=[
            pl.BlockSpec((BR, D), lambda i: (i, 0)),
            pl.BlockSpec((D, D), lambda i: (0, 0)),
            pl.BlockSpec((1, D), lambda i: (0, 0)),
        ],
        out_specs=pl.BlockSpec((BR, D), lambda i: (i, 0)),
        out_shape=jax.ShapeDtypeStruct((NN, D), jnp.float32),
    )(x, wix, bi2)


def _tc_h0p1(xs, ea, wie, wh):
    BR = 512

    def body(xs_ref, ea_ref, wie_ref, wh_ref, h0_ref, p_ref):
        h0 = xs_ref[...] + jnp.dot(ea_ref[...], wie_ref[...],
                                   preferred_element_type=jnp.float32)
        h0_ref[...] = h0
        p_ref[...] = jnp.dot(jnp.maximum(h0, 0.0), wh_ref[...],
                             preferred_element_type=jnp.float32)

    return pl.pallas_call(
        body,
        grid=(NE_P // BR,),
        in_specs=[
            pl.BlockSpec((BR, D), lambda i: (i, 0)),
            pl.BlockSpec((BR, DB), lambda i: (i, 0)),
            pl.BlockSpec((DB, D), lambda i: (0, 0)),
            pl.BlockSpec((D, D), lambda i: (0, 0)),
        ],
        out_specs=[pl.BlockSpec((BR, D), lambda i: (i, 0))] * 2,
        out_shape=[jax.ShapeDtypeStruct((NE_P, D), jnp.float32)] * 2,
    )(xs, ea, wie, wh)


def _tc_addp(parts):
    BR = 512

    def body(p_ref, o_ref):
        o_ref[...] = p_ref[0] + p_ref[1]

    return pl.pallas_call(
        body,
        grid=(NN_P // BR,),
        in_specs=[pl.BlockSpec((NC, BR, D), lambda i: (0, i, 0))],
        out_specs=pl.BlockSpec((BR, D), lambda i: (i, 0)),
        out_shape=jax.ShapeDtypeStruct((NN_P, D), jnp.float32),
    )(parts)


def _tc_mm2(h0, s_g, r_g, bh2, wh):
    BR = 512

    def body(h0_ref, s_ref, r_ref, b_ref, wh_ref, p_ref):
        a = h0_ref[...] + s_ref[...] - r_ref[...] + b_ref[...]
        p_ref[...] = jnp.dot(jnp.maximum(a, 0.0), wh_ref[...],
                             preferred_element_type=jnp.float32)

    return pl.pallas_call(
        body,
        grid=(NE_P // BR,),
        in_specs=[
            pl.BlockSpec((BR, D), lambda i: (i, 0)),
            pl.BlockSpec((BR, D), lambda i: (i, 0)),
            pl.BlockSpec((BR, D), lambda i: (i, 0)),
            pl.BlockSpec((1, D), lambda i: (0, 0)),
            pl.BlockSpec((D, D), lambda i: (0, 0)),
        ],
        out_specs=pl.BlockSpec((BR, D), lambda i: (i, 0)),
        out_shape=jax.ShapeDtypeStruct((NE_P, D), jnp.float32),
    )(h0, s_g, r_g, bh2, wh)


def _tc_h3(h0, s_g, r_g, bh2):
    BR = 512

    def body(h0_ref, s_ref, r_ref, b_ref, o_ref):
        a = h0_ref[...] + s_ref[...] - r_ref[...] + b_ref[...]
        o_ref[...] = jnp.maximum(a, 0.0)

    return pl.pallas_call(
        body,
        grid=(NE_P // BR,),
        in_specs=[
            pl.BlockSpec((BR, D), lambda i: (i, 0)),
            pl.BlockSpec((BR, D), lambda i: (i, 0)),
            pl.BlockSpec((BR, D), lambda i: (i, 0)),
            pl.BlockSpec((1, D), lambda i: (0, 0)),
        ],
        out_specs=pl.BlockSpec((BR, D), lambda i: (i, 0)),
        out_shape=jax.ShapeDtypeStruct((NE_P, D), jnp.float32),
    )(h0, s_g, r_g, bh2)


def _tc_out(x, parts, wox, wom, bo2):
    BR = 400

    def body(x_ref, p_ref, wox_ref, wom_ref, b_ref, o_ref):
        aggf = p_ref[0] + p_ref[1]
        rs = jnp.sum(aggf, axis=1, keepdims=True)
        xb = x_ref[...]
        mfin = jnp.where(rs == 0.0, xb, aggf)
        acc = jnp.dot(xb, wox_ref[...], preferred_element_type=jnp.float32)
        acc += jnp.dot(mfin, wom_ref[...], preferred_element_type=jnp.float32)
        o_ref[...] = jnp.maximum(acc + b_ref[...], 0.0)

    return pl.pallas_call(
        body,
        grid=(NN // BR,),
        in_specs=[
            pl.BlockSpec((BR, D), lambda i: (i, 0)),
            pl.BlockSpec((NC, BR, D), lambda i: (0, i, 0)),
            pl.BlockSpec((D, D), lambda i: (0, 0)),
            pl.BlockSpec((D, D), lambda i: (0, 0)),
            pl.BlockSpec((1, D), lambda i: (0, 0)),
        ],
        out_specs=pl.BlockSpec((BR, D), lambda i: (i, 0)),
        out_shape=jax.ShapeDtypeStruct((NN, D), jnp.float32),
    )(x, parts, wox, wom, bo2)


def kernel(x, edge_index, rev_edge_index, edge_attr, Wi, bi, Wh, bh, Wo, bo):
    src = edge_index[0]
    dst = edge_index[1]
    pad = NE_P - NE
    src3 = jnp.pad(src, (0, pad)).reshape(NW, NCH, CH)
    rev3 = jnp.pad(rev_edge_index, (0, pad)).reshape(NW, NCH, CH)
    dst3 = jnp.pad(dst, (0, pad), constant_values=NN_P - 1).reshape(NW, NCH, CH)
    ea_p = jnp.pad(edge_attr, ((0, pad), (0, 0)))
    zrows = jnp.zeros((NN_P, D), jnp.float32)
    wix, wie = Wi[:D], Wi[D:]
    wox, wom = Wo[:D], Wo[D:]
    bi2 = bi.reshape(1, D)
    bh2 = bh.reshape(1, D)
    bo2 = bo.reshape(1, D)

    g = _tc_g(x, wix, bi2)
    xs = _sc_gather(g, src3)
    h0, p = _tc_h0p1(xs, ea_p, wie, Wh)

    h3 = None
    for it in range(2):
        parts = _sc_scatter(p, dst3, zrows)
        agg = _tc_addp(parts)
        s_g = _sc_gather(agg, src3)
        r_g = _sc_gather(p, rev3)
        if it == 0:
            p = _tc_mm2(h0, s_g, r_g, bh2, Wh)
        else:
            h3 = _tc_h3(h0, s_g, r_g, bh2)

    parts = _sc_scatter(h3, dst3, zrows)
    return _tc_out(x, parts, wox, wom, bo2)


# speedup vs baseline: 1.1312x; 1.1312x over previous
import functools

import jax
import jax.numpy as jnp
from jax import lax
from jax.experimental import pallas as pl
from jax.experimental.pallas import tpu as pltpu
from jax.experimental.pallas import tpu_sc as plsc

NN = 10000
NE = 320000
D = 128
DB = 16
NC = 2
NS = 16
NW = NC * NS
CH = 128
NCH = 80
EPW = CH * NCH
NE_P = NW * EPW
NN_P = 10240
RPW = NN_P // NS
GB = 2
NG = NCH // GB
CPW = EPW // CH
NCR = NE_P // CH


def _mesh():
    return plsc.VectorSubcoreMesh(core_axis_name="c", subcore_axis_name="s")


def _sc_gather(table, idx3):
    @functools.partial(
        pl.kernel,
        out_type=jax.ShapeDtypeStruct((NCR, CH, D), jnp.float32),
        mesh=_mesh(),
        scratch_types=[
            pltpu.VMEM((NCH, CH), jnp.int32),
            pltpu.VMEM((2, GB, CH, D), jnp.float32),
            pltpu.SemaphoreType.DMA((2,)),
        ],
    )
    def k(table_hbm, idx_hbm, out_hbm, idx_v, rows_v, gsem):
        w = lax.axis_index("s") * NC + lax.axis_index("c")
        pltpu.sync_copy(idx_hbm.at[w], idx_v)
        cbase = w * CPW

        def g_chunk(g, b, b2):
            return pltpu.make_async_copy(
                table_hbm.at[idx_v.at[g * GB + b2]],
                rows_v.at[b, b2], gsem.at[b])

        def g_start(g, b):
            for b2 in range(GB):
                g_chunk(g, b, b2).start()

        def g_wait(g, b):
            for b2 in range(GB):
                g_chunk(g, b, b2).wait()

        g_start(0, 0)

        def body(t, carry):
            g0 = 2 * t

            @pl.when(g0 + 1 < NG)
            def _():
                g_start(g0 + 1, 1)

            g_wait(g0, 0)
            pltpu.sync_copy(rows_v.at[0],
                            out_hbm.at[pl.ds(cbase + g0 * GB, GB)])

            @pl.when(g0 + 2 < NG)
            def _():
                g_start(g0 + 2, 0)

            @pl.when(g0 + 1 < NG)
            def _():
                g_wait(g0 + 1, 1)
                pltpu.sync_copy(rows_v.at[1],
                                out_hbm.at[pl.ds(cbase + (g0 + 1) * GB, GB)])

            return carry

        lax.fori_loop(0, (NG + 1) // 2, body, 0)

    return k(table, idx3)


def _sc_scatter(vals3, dst3, zrows):
    @functools.partial(
        pl.kernel,
        out_type=jax.ShapeDtypeStruct((NC, NN_P, D), jnp.float32),
        mesh=_mesh(),
        scratch_types=[
            pltpu.VMEM((NCH, CH), jnp.int32),
            pltpu.VMEM((2, CH, D), jnp.float32),
            pltpu.VMEM_SHARED((NN_P, D), jnp.float32),
            pltpu.SemaphoreType.DMA((2,)),
        ],
    )
    def k(vals_hbm, idx_hbm, z_hbm, out_hbm, idx_v, rows_v, agg_sh, lsem):
        c = lax.axis_index("c")
        s = lax.axis_index("s")
        w = s * NC + c
        pltpu.sync_copy(z_hbm.at[pl.ds(s * RPW, RPW)],
                        agg_sh.at[pl.ds(s * RPW, RPW)])
        pltpu.sync_copy(idx_hbm.at[w], idx_v)
        plsc.subcore_barrier()
        cbase = w * CPW

        def l_copy(j, b):
            return pltpu.make_async_copy(
                vals_hbm.at[j + cbase], rows_v.at[b], lsem.at[b])

        def scat(j, b):
            pltpu.sync_copy(rows_v.at[b],
                            agg_sh.at[idx_v.at[j]], add=True)

        l_copy(0, 0).start()

        def body(t, carry):
            j0 = 2 * t

            @pl.when(j0 + 1 < NCH)
            def _():
                l_copy(j0 + 1, 1).start()

            l_copy(j0, 0).wait()
            scat(j0, 0)

            @pl.when(j0 + 2 < NCH)
            def _():
                l_copy(j0 + 2, 0).start()

            @pl.when(j0 + 1 < NCH)
            def _():
                l_copy(j0 + 1, 1).wait()
                scat(j0 + 1, 1)

            return carry

        lax.fori_loop(0, (NCH + 1) // 2, body, 0)
        plsc.subcore_barrier()
        pltpu.sync_copy(agg_sh.at[pl.ds(s * RPW, RPW)],
                        out_hbm.at[c, pl.ds(s * RPW, RPW)])

    return k(vals3, dst3, zrows)


def _tc_g(x, wix, bi2):
    BR = 400

    def body(x_ref, w_ref, b_ref, o_ref):
        o_ref[...] = jnp.dot(x_ref[...], w_ref[...],
                             preferred_element_type=jnp.float32) + b_ref[...]

    return pl.pallas_call(
        body,
        grid=(NN // BR,),
        in_specs=[
            pl.BlockSpec((BR, D), lambda i: (i, 0)),
            pl.BlockSpec((D, D), lambda i: (0, 0)),
            pl.BlockSpec((1, D), lambda i: (0, 0)),
        ],
        out_specs=pl.BlockSpec((BR, D), lambda i: (i, 0)),
        out_shape=jax.ShapeDtypeStruct((NN, D), jnp.float32),
    )(x, wix, bi2)


def _tc_h0p1(xs, ea, wie, wh):
    BR = 512

    def body(xs_ref, ea_ref, wie_ref, wh_ref, h0_ref, p_ref):
        h0 = xs_ref[...] + jnp.dot(ea_ref[...], wie_ref[...],
                                   preferred_element_type=jnp.float32)
        h0_ref[...] = h0
        p_ref[...] = jnp.dot(jnp.maximum(h0, 0.0), wh_ref[...],
                             preferred_element_type=jnp.float32)

    return pl.pallas_call(
        body,
        grid=(NE_P // BR,),
        in_specs=[
            pl.BlockSpec((BR, D), lambda i: (i, 0)),
            pl.BlockSpec((BR, DB), lambda i: (i, 0)),
            pl.BlockSpec((DB, D), lambda i: (0, 0)),
            pl.BlockSpec((D, D), lambda i: (0, 0)),
        ],
        out_specs=[pl.BlockSpec((BR, D), lambda i: (i, 0))] * 2,
        out_shape=[jax.ShapeDtypeStruct((NE_P, D), jnp.float32)] * 2,
    )(xs, ea, wie, wh)


def _tc_addp(parts):
    BR = 512

    def body(p_ref, o_ref):
        o_ref[...] = p_ref[0] + p_ref[1]

    return pl.pallas_call(
        body,
        grid=(NN_P // BR,),
        in_specs=[pl.BlockSpec((NC, BR, D), lambda i: (0, i, 0))],
        out_specs=pl.BlockSpec((BR, D), lambda i: (i, 0)),
        out_shape=jax.ShapeDtypeStruct((NN_P, D), jnp.float32),
    )(parts)


def _tc_mm2(h0, s_g, r_g, bh2, wh):
    BR = 512

    def body(h0_ref, s_ref, r_ref, b_ref, wh_ref, p_ref):
        a = h0_ref[...] + s_ref[...] - r_ref[...] + b_ref[...]
        p_ref[...] = jnp.dot(jnp.maximum(a, 0.0), wh_ref[...],
                             preferred_element_type=jnp.float32)

    return pl.pallas_call(
        body,
        grid=(NE_P // BR,),
        in_specs=[
            pl.BlockSpec((BR, D), lambda i: (i, 0)),
            pl.BlockSpec((BR, D), lambda i: (i, 0)),
            pl.BlockSpec((BR, D), lambda i: (i, 0)),
            pl.BlockSpec((1, D), lambda i: (0, 0)),
            pl.BlockSpec((D, D), lambda i: (0, 0)),
        ],
        out_specs=pl.BlockSpec((BR, D), lambda i: (i, 0)),
        out_shape=jax.ShapeDtypeStruct((NE_P, D), jnp.float32),
    )(h0, s_g, r_g, bh2, wh)


def _tc_h3(h0, s_g, r_g, bh2):
    BR = 512

    def body(h0_ref, s_ref, r_ref, b_ref, o_ref):
        a = h0_ref[...] + s_ref[...] - r_ref[...] + b_ref[...]
        o_ref[...] = jnp.maximum(a, 0.0)

    return pl.pallas_call(
        body,
        grid=(NE_P // BR,),
        in_specs=[
            pl.BlockSpec((BR, D), lambda i: (i, 0)),
            pl.BlockSpec((BR, D), lambda i: (i, 0)),
            pl.BlockSpec((BR, D), lambda i: (i, 0)),
            pl.BlockSpec((1, D), lambda i: (0, 0)),
        ],
        out_specs=pl.BlockSpec((BR, D), lambda i: (i, 0)),
        out_shape=jax.ShapeDtypeStruct((NE_P, D), jnp.float32),
    )(h0, s_g, r_g, bh2)


def _tc_out(x, parts, wox, wom, bo2):
    BR = 400

    def body(x_ref, p_ref, wox_ref, wom_ref, b_ref, o_ref):
        aggf = p_ref[0] + p_ref[1]
        rs = jnp.sum(aggf, axis=1, keepdims=True)
        xb = x_ref[...]
        mfin = jnp.where(rs == 0.0, xb, aggf)
        acc = jnp.dot(xb, wox_ref[...], preferred_element_type=jnp.float32)
        acc += jnp.dot(mfin, wom_ref[...], preferred_element_type=jnp.float32)
        o_ref[...] = jnp.maximum(acc + b_ref[...], 0.0)

    return pl.pallas_call(
        body,
        grid=(NN // BR,),
        in_specs=[
            pl.BlockSpec((BR, D), lambda i: (i, 0)),
            pl.BlockSpec((NC, BR, D), lambda i: (0, i, 0)),
            pl.BlockSpec((D, D), lambda i: (0, 0)),
            pl.BlockSpec((D, D), lambda i: (0, 0)),
            pl.BlockSpec((1, D), lambda i: (0, 0)),
        ],
        out_specs=pl.BlockSpec((BR, D), lambda i: (i, 0)),
        out_shape=jax.ShapeDtypeStruct((NN, D), jnp.float32),
    )(x, parts, wox, wom, bo2)


def kernel(x, edge_index, rev_edge_index, edge_attr, Wi, bi, Wh, bh, Wo, bo):
    src = edge_index[0]
    dst = edge_index[1]
    pad = NE_P - NE
    src3 = jnp.pad(src, (0, pad)).reshape(NW, NCH, CH)
    rev3 = jnp.pad(rev_edge_index, (0, pad)).reshape(NW, NCH, CH)
    dst3 = jnp.pad(dst, (0, pad), constant_values=NN_P - 1).reshape(NW, NCH, CH)
    ea_p = jnp.pad(edge_attr, ((0, pad), (0, 0)))
    zrows = jnp.zeros((NN_P, D), jnp.float32)
    wix, wie = Wi[:D], Wi[D:]
    wox, wom = Wo[:D], Wo[D:]
    bi2 = bi.reshape(1, D)
    bh2 = bh.reshape(1, D)
    bo2 = bo.reshape(1, D)

    flat = lambda a: a.reshape(NE_P, D)
    as3 = lambda a: a.reshape(NCR, CH, D)

    g = _tc_g(x, wix, bi2)
    xs = flat(_sc_gather(g, src3))
    h0, p = _tc_h0p1(xs, ea_p, wie, Wh)

    h3 = None
    for it in range(2):
        parts = _sc_scatter(as3(p), dst3, zrows)
        agg = _tc_addp(parts)
        s_g = flat(_sc_gather(agg, src3))
        r_g = flat(_sc_gather(p, rev3))
        if it == 0:
            p = _tc_mm2(h0, s_g, r_g, bh2, Wh)
        else:
            h3 = _tc_h3(h0, s_g, r_g, bh2)

    parts = _sc_scatter(as3(h3), dst3, zrows)
    return _tc_out(x, parts, wox, wom, bo2)
